# baseline (speedup 1.0000x reference)
import functools

import jax
import jax.numpy as jnp
from jax import lax
from jax.experimental import pallas as pl
from jax.experimental.pallas import tpu as pltpu
from jax.experimental.pallas import tpu_sc as plsc

NC, NS = 2, 16
NW = NC * NS
GCH = 128
TD = 256


def _sc_gather(table, idx, bn_pad):
    b_per_w = bn_pad // NW
    mesh = plsc.VectorSubcoreMesh(core_axis_name="c", subcore_axis_name="s")

    @functools.partial(
        pl.kernel,
        out_type=jax.ShapeDtypeStruct((bn_pad, TD), jnp.float32),
        mesh=mesh,
        scratch_types=[
            pltpu.VMEM((b_per_w,), jnp.int32),
            pltpu.VMEM((GCH, TD), jnp.float32),
            pltpu.VMEM((GCH, TD), jnp.float32),
            pltpu.SemaphoreType.DMA,
            pltpu.SemaphoreType.DMA,
            pltpu.SemaphoreType.DMA,
            pltpu.SemaphoreType.DMA,
        ],
    )
    def gather_kernel(t_hbm, idx_hbm, g_hbm, idx_v, r0, r1, sg0, sg1,
                      sw0, sw1):
        wid = lax.axis_index("s") * NC + lax.axis_index("c")
        base = wid * b_per_w
        pltpu.sync_copy(idx_hbm.at[pl.ds(base, b_per_w)], idx_v)
        npairs = b_per_w // (2 * GCH)

        def gcp(buf, sem, chunk):
            off = chunk * GCH
            return pltpu.make_async_copy(
                t_hbm.at[idx_v.at[pl.ds(off, GCH)]], buf, sem)

        def wcp(buf, sem, chunk):
            off = chunk * GCH
            return pltpu.make_async_copy(
                buf, g_hbm.at[pl.ds(base + off, GCH)], sem)

        gcp(r0, sg0, 0).start()
        gcp(r1, sg1, 1).start()

        @pl.loop(0, npairs - 1)
        def _(i):
            c0 = 2 * i
            gcp(r0, sg0, c0).wait()
            wcp(r0, sw0, c0).start()
            gcp(r1, sg1, c0 + 1).wait()
            wcp(r1, sw1, c0 + 1).start()
            wcp(r0, sw0, c0).wait()
            gcp(r0, sg0, c0 + 2).start()
            wcp(r1, sw1, c0 + 1).wait()
            gcp(r1, sg1, c0 + 3).start()

        last = 2 * (npairs - 1)
        gcp(r0, sg0, last).wait()
        wcp(r0, sw0, last).start()
        gcp(r1, sg1, last + 1).wait()
        wcp(r1, sw1, last + 1).start()
        wcp(r0, sw0, last).wait()
        wcp(r1, sw1, last + 1).wait()

    return gather_kernel(table, idx)


def _table_body(p0, p1, fl, o_ref):
    ii = lax.broadcasted_iota(jnp.int32, (64, 64), 0)
    jj = lax.broadcasted_iota(jnp.int32, (64, 64), 1)
    eye = (ii == jj).astype(jnp.float32)
    dn = (((0,), (0,)), ((), ()))

    def tr(m):
        x = jnp.reshape(m[0], (64, m.shape[2] * m.shape[3]))
        return lax.dot_general(x, eye, dn, preferred_element_type=jnp.float32)

    o_ref[:, 0:64] = tr(p0)
    o_ref[:, 64:128] = tr(p1)
    o_ref[:, 128:192] = tr(fl)


def _build_table(pc0, pc1, fl, b, C, HW):
    HB = 8
    T = HB * 512
    map_spec = pl.BlockSpec((1, C, HB, 512), lambda j: (b, 0, j, 0))
    return pl.pallas_call(
        _table_body,
        out_shape=jax.ShapeDtypeStruct((HW, TD), jnp.float32),
        grid=(HW // T,),
        in_specs=[map_spec, map_spec, map_spec],
        out_specs=pl.BlockSpec((T, TD), lambda j: (j, 0)),
        compiler_params=pltpu.CompilerParams(
            dimension_semantics=("parallel",)),
    )(pc0, pc1, fl)


def _gelu_exact(x):
    return 0.5 * x * (1.0 + lax.erf(x * 0.7071067811865476))


def _mlp_body(g, wl0, wl1, bl, w1a, w1b, b1r, w2r, b2r, w3r, b3r,
              w4r, b4r, o_ref):
    f32 = jnp.float32
    gb = g[...]
    g0, g1, g2 = gb[:, 0:64], gb[:, 64:128], gb[:, 128:192]
    x = (jnp.dot(g0, wl0[...], preferred_element_type=f32)
         + jnp.dot(g1, wl1[...], preferred_element_type=f32) + bl[...])
    h = (jnp.dot(x, w1a[...], preferred_element_type=f32)
         + jnp.dot(g2, w1b[...], preferred_element_type=f32) + b1r[...])
    h = _gelu_exact(h)
    h = _gelu_exact(jnp.dot(h, w2r[...], preferred_element_type=f32)
                    + b2r[...])
    h = _gelu_exact(jnp.dot(h, w3r[...], preferred_element_type=f32)
                    + b3r[...])
    s = jnp.dot(h, w4r[...], preferred_element_type=f32)[:, 0] + b4r[0]
    o_ref[...] = jax.nn.sigmoid(s)


def _mlp(g, N, weights):
    BLK = 512
    full = lambda shape: pl.BlockSpec(shape, lambda i: tuple(0 for _ in shape))
    return pl.pallas_call(
        _mlp_body,
        out_shape=jax.ShapeDtypeStruct((N,), jnp.float32),
        grid=(N // BLK,),
        in_specs=[pl.BlockSpec((BLK, TD), lambda i: (i, 0)),
                  full((64, 64)), full((64, 64)), full((64,)),
                  full((64, 64)), full((64, 64)), full((64,)),
                  full((64, 32)), full((32,)),
                  full((32, 16)), full((16,)),
                  full((16, 1)), full((1,))],
        out_specs=pl.BlockSpec((BLK,), lambda i: (i,)),
        compiler_params=pltpu.CompilerParams(
            dimension_semantics=("parallel",)),
    )(g, *weights)


def kernel(pc0_map, pc1_map, flow_map, lidar_voxel_coords, radar_voxel_coords,
           W_lin, b_lin, W1, b1, W2, b2, W3, b3, W4, b4):
    B, C, H, W = pc0_map.shape
    NL = lidar_voxel_coords.shape[1]
    NR = radar_voxel_coords.shape[1]
    N = NL + NR
    N_pad = -(-N // (NW * 2 * GCH)) * (NW * 2 * GCH)

    coords = jnp.concatenate([lidar_voxel_coords, radar_voxel_coords], axis=1)
    idx = (coords[..., 1] * W + coords[..., 2]).astype(jnp.int32)
    idx = jnp.pad(idx, ((0, 0), (0, N_pad - N)))

    weights = (W_lin[:64], W_lin[64:], b_lin, W1[:64], W1[64:], b1,
               W2, b2, W3, b3, W4, b4)

    outs = []
    for b in range(B):
        table = _build_table(pc0_map, pc1_map, flow_map, b, C, H * W)
        g = _sc_gather(table, idx[b], N_pad)
        outs.append(_mlp(g, N, weights))
    return jnp.stack(outs, axis=0)


# speedup vs baseline: 1.0298x; 1.0298x over previous
import functools

import jax
import jax.numpy as jnp
from jax import lax
from jax.experimental import pallas as pl
from jax.experimental.pallas import tpu as pltpu
from jax.experimental.pallas import tpu_sc as plsc

NC, NS = 2, 16
NW = NC * NS
GCH = 128
TD = 256
TDP = 128


def _sc_gather(table, idx, bn_pad):
    b_per_w = bn_pad // NW
    mesh = plsc.VectorSubcoreMesh(core_axis_name="c", subcore_axis_name="s")

    @functools.partial(
        pl.kernel,
        out_type=jax.ShapeDtypeStruct((bn_pad, TDP), jnp.int32),
        mesh=mesh,
        scratch_types=[
            pltpu.VMEM((b_per_w,), jnp.int32),
            pltpu.VMEM((GCH, TDP), jnp.int32),
            pltpu.VMEM((GCH, TDP), jnp.int32),
            pltpu.SemaphoreType.DMA,
            pltpu.SemaphoreType.DMA,
            pltpu.SemaphoreType.DMA,
            pltpu.SemaphoreType.DMA,
        ],
    )
    def gather_kernel(t_hbm, idx_hbm, g_hbm, idx_v, r0, r1, sg0, sg1,
                      sw0, sw1):
        wid = lax.axis_index("s") * NC + lax.axis_index("c")
        base = wid * b_per_w
        pltpu.sync_copy(idx_hbm.at[pl.ds(base, b_per_w)], idx_v)
        npairs = b_per_w // (2 * GCH)

        def gcp(buf, sem, chunk):
            off = chunk * GCH
            return pltpu.make_async_copy(
                t_hbm.at[idx_v.at[pl.ds(off, GCH)]], buf, sem)

        def wcp(buf, sem, chunk):
            off = chunk * GCH
            return pltpu.make_async_copy(
                buf, g_hbm.at[pl.ds(base + off, GCH)], sem)

        gcp(r0, sg0, 0).start()
        gcp(r1, sg1, 1).start()

        @pl.loop(0, npairs - 1)
        def _(i):
            c0 = 2 * i
            gcp(r0, sg0, c0).wait()
            wcp(r0, sw0, c0).start()
            gcp(r1, sg1, c0 + 1).wait()
            wcp(r1, sw1, c0 + 1).start()
            wcp(r0, sw0, c0).wait()
            gcp(r0, sg0, c0 + 2).start()
            wcp(r1, sw1, c0 + 1).wait()
            gcp(r1, sg1, c0 + 3).start()

        last = 2 * (npairs - 1)
        gcp(r0, sg0, last).wait()
        wcp(r0, sw0, last).start()
        gcp(r1, sg1, last + 1).wait()
        wcp(r1, sw1, last + 1).start()
        wcp(r0, sw0, last).wait()
        wcp(r1, sw1, last + 1).wait()

    return gather_kernel(table, idx)


def _table_body(p0, p1, fl, o_ref):
    ii = lax.broadcasted_iota(jnp.int32, (64, 32), 0)
    jj = lax.broadcasted_iota(jnp.int32, (64, 32), 1)
    e_lo = (ii == jj).astype(jnp.float32)
    e_hi = (ii == jj + 32).astype(jnp.float32)
    dn = (((0,), (0,)), ((), ()))
    mask_hi = jnp.int32(-65536)

    def pack(m):
        x = jnp.reshape(m[0], (64, m.shape[2] * m.shape[3]))
        lo = lax.dot_general(x, e_lo, dn, preferred_element_type=jnp.float32)
        hi = lax.dot_general(x, e_hi, dn, preferred_element_type=jnp.float32)
        lo_b = lax.bitcast_convert_type(
            lo.astype(jnp.bfloat16).astype(jnp.float32), jnp.int32)
        hi_b = lax.bitcast_convert_type(
            hi.astype(jnp.bfloat16).astype(jnp.float32), jnp.int32)
        return lax.shift_right_logical(lo_b, 16) | (hi_b & mask_hi)

    o_ref[:, 0:32] = pack(p0)
    o_ref[:, 32:64] = pack(p1)
    o_ref[:, 64:96] = pack(fl)


def _build_table(pc0, pc1, fl, b, C, HW):
    HB = 8
    T = HB * 512
    map_spec = pl.BlockSpec((1, C, HB, 512), lambda j: (b, 0, j, 0))
    return pl.pallas_call(
        _table_body,
        out_shape=jax.ShapeDtypeStruct((HW, TDP), jnp.int32),
        grid=(HW // T,),
        in_specs=[map_spec, map_spec, map_spec],
        out_specs=pl.BlockSpec((T, TD), lambda j: (j, 0)),
        compiler_params=pltpu.CompilerParams(
            dimension_semantics=("parallel",)),
    )(pc0, pc1, fl)


def _gelu_exact(x):
    return 0.5 * x * (1.0 + lax.erf(x * 0.7071067811865476))


def _mlp_body(g, wl0, wl1, bl, w1a, w1b, b1r, w2r, b2r, w3r, b3r,
              w4r, b4r, o_ref):
    f32 = jnp.float32
    gb = g[...]

    def unpack(s):
        lo = lax.bitcast_convert_type(jnp.left_shift(s, 16), f32)
        hi = lax.bitcast_convert_type(s & jnp.int32(-65536), f32)
        return jnp.concatenate([lo, hi], axis=1).astype(jnp.bfloat16)

    g0 = unpack(gb[:, 0:32])
    g1 = unpack(gb[:, 32:64])
    g2 = unpack(gb[:, 64:96])
    x = (jnp.dot(g0, wl0[...], preferred_element_type=f32)
         + jnp.dot(g1, wl1[...], preferred_element_type=f32) + bl[...])
    h = (jnp.dot(x.astype(jnp.bfloat16), w1a[...],
                 preferred_element_type=f32)
         + jnp.dot(g2, w1b[...], preferred_element_type=f32) + b1r[...])
    h = _gelu_exact(h)
    h = _gelu_exact(jnp.dot(h, w2r[...], preferred_element_type=f32)
                    + b2r[...])
    h = _gelu_exact(jnp.dot(h, w3r[...], preferred_element_type=f32)
                    + b3r[...])
    s = jnp.dot(h, w4r[...], preferred_element_type=f32)[:, 0] + b4r[0]
    o_ref[...] = jax.nn.sigmoid(s)


def _mlp(g, N, weights):
    BLK = 512
    full = lambda shape: pl.BlockSpec(shape, lambda i: tuple(0 for _ in shape))
    return pl.pallas_call(
        _mlp_body,
        out_shape=jax.ShapeDtypeStruct((N,), jnp.float32),
        grid=(N // BLK,),
        in_specs=[pl.BlockSpec((BLK, TDP), lambda i: (i, 0)),
                  full((64, 64)), full((64, 64)), full((64,)),
                  full((64, 64)), full((64, 64)), full((64,)),
                  full((64, 32)), full((32,)),
                  full((32, 16)), full((16,)),
                  full((16, 1)), full((1,))],
        out_specs=pl.BlockSpec((BLK,), lambda i: (i,)),
        compiler_params=pltpu.CompilerParams(
            dimension_semantics=("parallel",)),
    )(g, *weights)


def kernel(pc0_map, pc1_map, flow_map, lidar_voxel_coords, radar_voxel_coords,
           W_lin, b_lin, W1, b1, W2, b2, W3, b3, W4, b4):
    B, C, H, W = pc0_map.shape
    NL = lidar_voxel_coords.shape[1]
    NR = radar_voxel_coords.shape[1]
    N = NL + NR
    N_pad = -(-N // (NW * 2 * GCH)) * (NW * 2 * GCH)

    coords = jnp.concatenate([lidar_voxel_coords, radar_voxel_coords], axis=1)
    idx = (coords[..., 1] * W + coords[..., 2]).astype(jnp.int32)
    idx = jnp.pad(idx, ((0, 0), (0, N_pad - N)))

    bf16 = jnp.bfloat16
    weights = (W_lin[:64].astype(bf16), W_lin[64:].astype(bf16), b_lin,
               W1[:64].astype(bf16), W1[64:].astype(bf16), b1,
               W2, b2, W3, b3, W4, b4)

    outs = []
    for b in range(B):
        table = _build_table(pc0_map, pc1_map, flow_map, b, C, H * W)
        g = _sc_gather(table, idx[b], N_pad)
        outs.append(_mlp(g, N, weights))
    return jnp.stack(outs, axis=0)


# speedup vs baseline: 1.1626x; 1.1290x over previous
import functools

import jax
import jax.numpy as jnp
from jax import lax
from jax.experimental import pallas as pl
from jax.experimental.pallas import tpu as pltpu
from jax.experimental.pallas import tpu_sc as plsc

NC, NS = 2, 16
NW = NC * NS
GCH = 128
TD = 256
TDP = 128


def _sc_gather(table, idx, bn_pad):
    b_per_w = bn_pad // NW
    mesh = plsc.VectorSubcoreMesh(core_axis_name="c", subcore_axis_name="s")

    @functools.partial(
        pl.kernel,
        out_type=jax.ShapeDtypeStruct((bn_pad, TDP), jnp.int32),
        mesh=mesh,
        scratch_types=[
            pltpu.VMEM((b_per_w,), jnp.int32),
            pltpu.VMEM((GCH, TDP), jnp.int32),
            pltpu.VMEM((GCH, TDP), jnp.int32),
            pltpu.SemaphoreType.DMA,
            pltpu.SemaphoreType.DMA,
            pltpu.SemaphoreType.DMA,
            pltpu.SemaphoreType.DMA,
        ],
    )
    def gather_kernel(t_hbm, idx_hbm, g_hbm, idx_v, r0, r1, sg0, sg1,
                      sw0, sw1):
        wid = lax.axis_index("s") * NC + lax.axis_index("c")
        base = wid * b_per_w
        pltpu.sync_copy(idx_hbm.at[pl.ds(base, b_per_w)], idx_v)
        npairs = b_per_w // (2 * GCH)

        def gcp(buf, sem, chunk):
            off = chunk * GCH
            return pltpu.make_async_copy(
                t_hbm.at[idx_v.at[pl.ds(off, GCH)]], buf, sem)

        def wcp(buf, sem, chunk):
            off = chunk * GCH
            return pltpu.make_async_copy(
                buf, g_hbm.at[pl.ds(base + off, GCH)], sem)

        gcp(r0, sg0, 0).start()
        gcp(r1, sg1, 1).start()

        @pl.loop(0, npairs - 1)
        def _(i):
            c0 = 2 * i
            gcp(r0, sg0, c0).wait()
            wcp(r0, sw0, c0).start()
            gcp(r1, sg1, c0 + 1).wait()
            wcp(r1, sw1, c0 + 1).start()
            wcp(r0, sw0, c0).wait()
            gcp(r0, sg0, c0 + 2).start()
            wcp(r1, sw1, c0 + 1).wait()
            gcp(r1, sg1, c0 + 3).start()

        last = 2 * (npairs - 1)
        gcp(r0, sg0, last).wait()
        wcp(r0, sw0, last).start()
        gcp(r1, sg1, last + 1).wait()
        wcp(r1, sw1, last + 1).start()
        wcp(r0, sw0, last).wait()
        wcp(r1, sw1, last + 1).wait()

    return gather_kernel(table, idx)


def _table_body(p0, p1, fl, o_ref):
    ii = lax.broadcasted_iota(jnp.int32, (64, 32), 0)
    jj = lax.broadcasted_iota(jnp.int32, (64, 32), 1)
    e_lo = (ii == jj).astype(jnp.bfloat16)
    e_hi = (ii == jj + 32).astype(jnp.bfloat16)
    dn = (((0,), (0,)), ((), ()))
    mask_hi = jnp.int32(-65536)

    def pack(m):
        x = m[0].astype(jnp.bfloat16)
        x = jnp.reshape(x, (64, m.shape[2] * m.shape[3]))
        lo = lax.dot_general(x, e_lo, dn, preferred_element_type=jnp.float32)
        hi = lax.dot_general(x, e_hi, dn, preferred_element_type=jnp.float32)
        lo_b = lax.bitcast_convert_type(lo, jnp.int32)
        hi_b = lax.bitcast_convert_type(hi, jnp.int32)
        return lax.shift_right_logical(lo_b, 16) | (hi_b & mask_hi)

    o_ref[:, 0:32] = pack(p0)
    o_ref[:, 32:64] = pack(p1)
    o_ref[:, 64:96] = pack(fl)


def _build_table(pc0, pc1, fl, b, C, HW):
    HB = 8
    T = HB * 512
    map_spec = pl.BlockSpec((1, C, HB, 512), lambda j: (b, 0, j, 0))
    return pl.pallas_call(
        _table_body,
        out_shape=jax.ShapeDtypeStruct((HW, TDP), jnp.int32),
        grid=(HW // T,),
        in_specs=[map_spec, map_spec, map_spec],
        out_specs=pl.BlockSpec((T, TD), lambda j: (j, 0)),
        compiler_params=pltpu.CompilerParams(
            dimension_semantics=("parallel",)),
    )(pc0, pc1, fl)


def _gelu_exact(x):
    return 0.5 * x * (1.0 + lax.erf(x * 0.7071067811865476))


def _mlp_body(g, wl0, wl1, bl, w1a, w1b, b1r, w2r, b2r, w3r, b3r,
              w4r, b4r, o_ref):
    f32 = jnp.float32
    gb = g[...]

    def unpack(s):
        lo = lax.bitcast_convert_type(jnp.left_shift(s, 16), f32)
        hi = lax.bitcast_convert_type(s & jnp.int32(-65536), f32)
        return jnp.concatenate([lo, hi], axis=1).astype(jnp.bfloat16)

    g0 = unpack(gb[:, 0:32])
    g1 = unpack(gb[:, 32:64])
    g2 = unpack(gb[:, 64:96])
    x = (jnp.dot(g0, wl0[...], preferred_element_type=f32)
         + jnp.dot(g1, wl1[...], preferred_element_type=f32) + bl[...])
    h = (jnp.dot(x.astype(jnp.bfloat16), w1a[...],
                 preferred_element_type=f32)
         + jnp.dot(g2, w1b[...], preferred_element_type=f32) + b1r[...])
    h = _gelu_exact(h)
    h = _gelu_exact(jnp.dot(h, w2r[...], preferred_element_type=f32)
                    + b2r[...])
    h = _gelu_exact(jnp.dot(h, w3r[...], preferred_element_type=f32)
                    + b3r[...])
    s = jnp.sum(h * w4r[...][:, 0], axis=1) + b4r[0]
    o_ref[...] = jax.nn.sigmoid(s)


def _mlp(g, N, weights):
    BLK = 512
    full = lambda shape: pl.BlockSpec(shape, lambda i: tuple(0 for _ in shape))
    return pl.pallas_call(
        _mlp_body,
        out_shape=jax.ShapeDtypeStruct((N,), jnp.float32),
        grid=(N // BLK,),
        in_specs=[pl.BlockSpec((BLK, TDP), lambda i: (i, 0)),
                  full((64, 64)), full((64, 64)), full((64,)),
                  full((64, 64)), full((64, 64)), full((64,)),
                  full((64, 32)), full((32,)),
                  full((32, 16)), full((16,)),
                  full((16, 1)), full((1,))],
        out_specs=pl.BlockSpec((BLK,), lambda i: (i,)),
        compiler_params=pltpu.CompilerParams(
            dimension_semantics=("parallel",)),
    )(g, *weights)


def kernel(pc0_map, pc1_map, flow_map, lidar_voxel_coords, radar_voxel_coords,
           W_lin, b_lin, W1, b1, W2, b2, W3, b3, W4, b4):
    B, C, H, W = pc0_map.shape
    NL = lidar_voxel_coords.shape[1]
    NR = radar_voxel_coords.shape[1]
    N = NL + NR
    N_pad = -(-N // (NW * 2 * GCH)) * (NW * 2 * GCH)

    coords = jnp.concatenate([lidar_voxel_coords, radar_voxel_coords], axis=1)
    idx = (coords[..., 1] * W + coords[..., 2]).astype(jnp.int32)
    idx = jnp.pad(idx, ((0, 0), (0, N_pad - N)))

    bf16 = jnp.bfloat16
    weights = (W_lin[:64].astype(bf16), W_lin[64:].astype(bf16), b_lin,
               W1[:64].astype(bf16), W1[64:].astype(bf16), b1,
               W2, b2, W3, b3, W4, b4)

    outs = []
    for b in range(B):
        table = _build_table(pc0_map, pc1_map, flow_map, b, C, H * W)
        g = _sc_gather(table, idx[b], N_pad)
        outs.append(_mlp(g, N, weights))
    return jnp.stack(outs, axis=0)


# speedup vs baseline: 1.1685x; 1.0050x over previous
import functools

import jax
import jax.numpy as jnp
from jax import lax
from jax.experimental import pallas as pl
from jax.experimental.pallas import tpu as pltpu
from jax.experimental.pallas import tpu_sc as plsc

NC, NS = 2, 16
NW = NC * NS
GCH = 256
TD = 256
TDP = 128


def _sc_gather(table, idx, bn_pad):
    b_per_w = bn_pad // NW
    mesh = plsc.VectorSubcoreMesh(core_axis_name="c", subcore_axis_name="s")

    @functools.partial(
        pl.kernel,
        out_type=jax.ShapeDtypeStruct((bn_pad, TDP), jnp.int32),
        mesh=mesh,
        scratch_types=[
            pltpu.VMEM((b_per_w,), jnp.int32),
            pltpu.VMEM((GCH, TDP), jnp.int32),
            pltpu.VMEM((GCH, TDP), jnp.int32),
            pltpu.SemaphoreType.DMA,
            pltpu.SemaphoreType.DMA,
            pltpu.SemaphoreType.DMA,
            pltpu.SemaphoreType.DMA,
        ],
    )
    def gather_kernel(t_hbm, idx_hbm, g_hbm, idx_v, r0, r1, sg0, sg1,
                      sw0, sw1):
        wid = lax.axis_index("s") * NC + lax.axis_index("c")
        base = wid * b_per_w
        pltpu.sync_copy(idx_hbm.at[pl.ds(base, b_per_w)], idx_v)
        npairs = b_per_w // (2 * GCH)

        def gcp(buf, sem, chunk):
            off = chunk * GCH
            return pltpu.make_async_copy(
                t_hbm.at[idx_v.at[pl.ds(off, GCH)]], buf, sem)

        def wcp(buf, sem, chunk):
            off = chunk * GCH
            return pltpu.make_async_copy(
                buf, g_hbm.at[pl.ds(base + off, GCH)], sem)

        gcp(r0, sg0, 0).start()
        gcp(r1, sg1, 1).start()

        @pl.loop(0, npairs - 1)
        def _(i):
            c0 = 2 * i
            gcp(r0, sg0, c0).wait()
            wcp(r0, sw0, c0).start()
            gcp(r1, sg1, c0 + 1).wait()
            wcp(r1, sw1, c0 + 1).start()
            wcp(r0, sw0, c0).wait()
            gcp(r0, sg0, c0 + 2).start()
            wcp(r1, sw1, c0 + 1).wait()
            gcp(r1, sg1, c0 + 3).start()

        last = 2 * (npairs - 1)
        gcp(r0, sg0, last).wait()
        wcp(r0, sw0, last).start()
        gcp(r1, sg1, last + 1).wait()
        wcp(r1, sw1, last + 1).start()
        wcp(r0, sw0, last).wait()
        wcp(r1, sw1, last + 1).wait()

    return gather_kernel(table, idx)


def _table_body(p0, p1, fl, o_ref):
    ii = lax.broadcasted_iota(jnp.int32, (64, 32), 0)
    jj = lax.broadcasted_iota(jnp.int32, (64, 32), 1)
    e_lo = (ii == jj).astype(jnp.bfloat16)
    e_hi = (ii == jj + 32).astype(jnp.bfloat16)
    dn = (((0,), (0,)), ((), ()))
    mask_hi = jnp.int32(-65536)

    def pack(m):
        x = m[0].astype(jnp.bfloat16)
        x = jnp.reshape(x, (64, m.shape[2] * m.shape[3]))
        lo = lax.dot_general(x, e_lo, dn, preferred_element_type=jnp.float32)
        hi = lax.dot_general(x, e_hi, dn, preferred_element_type=jnp.float32)
        lo_b = lax.bitcast_convert_type(lo, jnp.int32)
        hi_b = lax.bitcast_convert_type(hi, jnp.int32)
        return lax.shift_right_logical(lo_b, 16) | (hi_b & mask_hi)

    o_ref[:, 0:32] = pack(p0)
    o_ref[:, 32:64] = pack(p1)
    o_ref[:, 64:96] = pack(fl)


def _build_table(pc0, pc1, fl, b, C, HW):
    HB = 8
    T = HB * 512
    map_spec = pl.BlockSpec((1, C, HB, 512), lambda j: (b, 0, j, 0))
    return pl.pallas_call(
        _table_body,
        out_shape=jax.ShapeDtypeStruct((HW, TDP), jnp.int32),
        grid=(HW // T,),
        in_specs=[map_spec, map_spec, map_spec],
        out_specs=pl.BlockSpec((T, TD), lambda j: (j, 0)),
        compiler_params=pltpu.CompilerParams(
            dimension_semantics=("parallel",)),
    )(pc0, pc1, fl)


def _gelu_exact(x):
    return 0.5 * x * (1.0 + lax.erf(x * 0.7071067811865476))


def _mlp_body(g, wl0, wl1, bl, w1a, w1b, b1r, w2r, b2r, w3r, b3r,
              w4r, b4r, o_ref):
    f32 = jnp.float32
    gb = g[...]

    def unpack(s):
        lo = lax.bitcast_convert_type(jnp.left_shift(s, 16), f32)
        hi = lax.bitcast_convert_type(s & jnp.int32(-65536), f32)
        return jnp.concatenate([lo, hi], axis=1).astype(jnp.bfloat16)

    g0 = unpack(gb[:, 0:32])
    g1 = unpack(gb[:, 32:64])
    g2 = unpack(gb[:, 64:96])
    x = (jnp.dot(g0, wl0[...], preferred_element_type=f32)
         + jnp.dot(g1, wl1[...], preferred_element_type=f32) + bl[...])
    h = (jnp.dot(x.astype(jnp.bfloat16), w1a[...],
                 preferred_element_type=f32)
         + jnp.dot(g2, w1b[...], preferred_element_type=f32) + b1r[...])
    h = _gelu_exact(h)
    h = _gelu_exact(jnp.dot(h, w2r[...], preferred_element_type=f32)
                    + b2r[...])
    h = _gelu_exact(jnp.dot(h, w3r[...], preferred_element_type=f32)
                    + b3r[...])
    s = jnp.sum(h * w4r[...][:, 0], axis=1) + b4r[0]
    o_ref[...] = jax.nn.sigmoid(s)


def _mlp(g, N, weights):
    BLK = 512
    full = lambda shape: pl.BlockSpec(shape, lambda i: tuple(0 for _ in shape))
    return pl.pallas_call(
        _mlp_body,
        out_shape=jax.ShapeDtypeStruct((N,), jnp.float32),
        grid=(N // BLK,),
        in_specs=[pl.BlockSpec((BLK, TDP), lambda i: (i, 0)),
                  full((64, 64)), full((64, 64)), full((64,)),
                  full((64, 64)), full((64, 64)), full((64,)),
                  full((64, 32)), full((32,)),
                  full((32, 16)), full((16,)),
                  full((16, 1)), full((1,))],
        out_specs=pl.BlockSpec((BLK,), lambda i: (i,)),
        compiler_params=pltpu.CompilerParams(
            dimension_semantics=("parallel",)),
    )(g, *weights)


def kernel(pc0_map, pc1_map, flow_map, lidar_voxel_coords, radar_voxel_coords,
           W_lin, b_lin, W1, b1, W2, b2, W3, b3, W4, b4):
    B, C, H, W = pc0_map.shape
    NL = lidar_voxel_coords.shape[1]
    NR = radar_voxel_coords.shape[1]
    N = NL + NR
    N_pad = -(-N // (NW * 2 * GCH)) * (NW * 2 * GCH)

    coords = jnp.concatenate([lidar_voxel_coords, radar_voxel_coords], axis=1)
    idx = (coords[..., 1] * W + coords[..., 2]).astype(jnp.int32)
    idx = jnp.pad(idx, ((0, 0), (0, N_pad - N)))

    bf16 = jnp.bfloat16
    weights = (W_lin[:64].astype(bf16), W_lin[64:].astype(bf16), b_lin,
               W1[:64].astype(bf16), W1[64:].astype(bf16), b1,
               W2, b2, W3, b3, W4, b4)

    outs = []
    for b in range(B):
        table = _build_table(pc0_map, pc1_map, flow_map, b, C, H * W)
        g = _sc_gather(table, idx[b], N_pad)
        outs.append(_mlp(g, N, weights))
    return jnp.stack(outs, axis=0)


# speedup vs baseline: 1.2231x; 1.0467x over previous
import functools

import jax
import jax.numpy as jnp
from jax import lax
from jax.experimental import pallas as pl
from jax.experimental.pallas import tpu as pltpu
from jax.experimental.pallas import tpu_sc as plsc

NC, NS = 2, 16
NW = NC * NS
GCH = 256
TD = 256
TDP = 128


def _sc_gather(table, idx, bn_pad):
    b_per_w = bn_pad // NW
    mesh = plsc.VectorSubcoreMesh(core_axis_name="c", subcore_axis_name="s")

    @functools.partial(
        pl.kernel,
        out_type=jax.ShapeDtypeStruct((bn_pad, TDP), jnp.int32),
        mesh=mesh,
        scratch_types=[
            pltpu.VMEM((b_per_w,), jnp.int32),
            pltpu.VMEM((GCH, TDP), jnp.int32),
            pltpu.VMEM((GCH, TDP), jnp.int32),
            pltpu.SemaphoreType.DMA,
            pltpu.SemaphoreType.DMA,
            pltpu.SemaphoreType.DMA,
            pltpu.SemaphoreType.DMA,
        ],
    )
    def gather_kernel(t_hbm, idx_hbm, g_hbm, idx_v, r0, r1, sg0, sg1,
                      sw0, sw1):
        wid = lax.axis_index("s") * NC + lax.axis_index("c")
        base = wid * b_per_w
        pltpu.sync_copy(idx_hbm.at[pl.ds(base, b_per_w)], idx_v)
        npairs = b_per_w // (2 * GCH)

        def gcp(buf, sem, chunk):
            off = chunk * GCH
            return pltpu.make_async_copy(
                t_hbm.at[idx_v.at[pl.ds(off, GCH)]], buf, sem)

        def wcp(buf, sem, chunk):
            off = chunk * GCH
            return pltpu.make_async_copy(
                buf, g_hbm.at[pl.ds(base + off, GCH)], sem)

        gcp(r0, sg0, 0).start()
        gcp(r1, sg1, 1).start()

        @pl.loop(0, npairs - 1)
        def _(i):
            c0 = 2 * i
            gcp(r0, sg0, c0).wait()
            wcp(r0, sw0, c0).start()
            gcp(r1, sg1, c0 + 1).wait()
            wcp(r1, sw1, c0 + 1).start()
            wcp(r0, sw0, c0).wait()
            gcp(r0, sg0, c0 + 2).start()
            wcp(r1, sw1, c0 + 1).wait()
            gcp(r1, sg1, c0 + 3).start()

        last = 2 * (npairs - 1)
        gcp(r0, sg0, last).wait()
        wcp(r0, sw0, last).start()
        gcp(r1, sg1, last + 1).wait()
        wcp(r1, sw1, last + 1).start()
        wcp(r0, sw0, last).wait()
        wcp(r1, sw1, last + 1).wait()

    return gather_kernel(table, idx)


def _table_body(p0, p1, fl, o_ref):
    bf = jnp.bfloat16
    xs = [m[0].astype(bf) for m in (p0, p1, fl)]
    X = jnp.concatenate(xs, axis=0)
    X = jnp.reshape(X, (192, X.shape[1] * X.shape[2]))
    ci = lax.broadcasted_iota(jnp.int32, (192, 128), 0)
    lj = lax.broadcasted_iota(jnp.int32, (192, 128), 1)
    p_ = ci // 64
    ch = ci % 64
    e_lo = ((ch < 32) & (lj == p_ * 32 + ch)).astype(bf)
    e_hi = ((ch >= 32) & (lj == p_ * 32 + ch - 32)).astype(bf)
    dn = (((0,), (0,)), ((), ()))
    lo = lax.dot_general(X, e_lo, dn, preferred_element_type=jnp.float32)
    hi = lax.dot_general(X, e_hi, dn, preferred_element_type=jnp.float32)
    lo_b = lax.bitcast_convert_type(lo, jnp.int32)
    hi_b = lax.bitcast_convert_type(hi, jnp.int32)
    o_ref[...] = (lax.shift_right_logical(lo_b, 16)
                  | (hi_b & jnp.int32(-65536)))


def _build_table(pc0, pc1, fl, b, C, HW):
    HB = 8
    T = HB * 512
    map_spec = pl.BlockSpec((1, C, HB, 512), lambda j: (b, 0, j, 0))
    return pl.pallas_call(
        _table_body,
        out_shape=jax.ShapeDtypeStruct((HW, TDP), jnp.int32),
        grid=(HW // T,),
        in_specs=[map_spec, map_spec, map_spec],
        out_specs=pl.BlockSpec((T, TDP), lambda j: (j, 0)),
        compiler_params=pltpu.CompilerParams(
            dimension_semantics=("parallel",)),
    )(pc0, pc1, fl)


def _gelu_exact(x):
    return 0.5 * x * (1.0 + lax.erf(x * 0.7071067811865476))


def _mlp_body(g, wl0, wl1, bl, w1a, w1b, b1r, w2r, b2r, w3r, b3r,
              w4r, b4r, o_ref):
    f32 = jnp.float32
    gb = g[...]

    def unpack(s):
        lo = lax.bitcast_convert_type(jnp.left_shift(s, 16), f32)
        hi = lax.bitcast_convert_type(s & jnp.int32(-65536), f32)
        return jnp.concatenate([lo, hi], axis=1).astype(jnp.bfloat16)

    g0 = unpack(gb[:, 0:32])
    g1 = unpack(gb[:, 32:64])
    g2 = unpack(gb[:, 64:96])
    x = (jnp.dot(g0, wl0[...], preferred_element_type=f32)
         + jnp.dot(g1, wl1[...], preferred_element_type=f32) + bl[...])
    h = (jnp.dot(x.astype(jnp.bfloat16), w1a[...],
                 preferred_element_type=f32)
         + jnp.dot(g2, w1b[...], preferred_element_type=f32) + b1r[...])
    h = _gelu_exact(h)
    h = _gelu_exact(jnp.dot(h, w2r[...], preferred_element_type=f32)
                    + b2r[...])
    h = _gelu_exact(jnp.dot(h, w3r[...], preferred_element_type=f32)
                    + b3r[...])
    s = jnp.sum(h * w4r[...][:, 0], axis=1) + b4r[0]
    o_ref[...] = jax.nn.sigmoid(s)


def _mlp(g, N, weights):
    BLK = 512
    full = lambda shape: pl.BlockSpec(shape, lambda i: tuple(0 for _ in shape))
    return pl.pallas_call(
        _mlp_body,
        out_shape=jax.ShapeDtypeStruct((N,), jnp.float32),
        grid=(N // BLK,),
        in_specs=[pl.BlockSpec((BLK, TDP), lambda i: (i, 0)),
                  full((64, 64)), full((64, 64)), full((64,)),
                  full((64, 64)), full((64, 64)), full((64,)),
                  full((64, 32)), full((32,)),
                  full((32, 16)), full((16,)),
                  full((16, 1)), full((1,))],
        out_specs=pl.BlockSpec((BLK,), lambda i: (i,)),
        compiler_params=pltpu.CompilerParams(
            dimension_semantics=("parallel",)),
    )(g, *weights)


def kernel(pc0_map, pc1_map, flow_map, lidar_voxel_coords, radar_voxel_coords,
           W_lin, b_lin, W1, b1, W2, b2, W3, b3, W4, b4):
    B, C, H, W = pc0_map.shape
    NL = lidar_voxel_coords.shape[1]
    NR = radar_voxel_coords.shape[1]
    N = NL + NR
    N_pad = -(-N // (NW * 2 * GCH)) * (NW * 2 * GCH)

    coords = jnp.concatenate([lidar_voxel_coords, radar_voxel_coords], axis=1)
    idx = (coords[..., 1] * W + coords[..., 2]).astype(jnp.int32)
    idx = jnp.pad(idx, ((0, 0), (0, N_pad - N)))

    bf16 = jnp.bfloat16
    weights = (W_lin[:64].astype(bf16), W_lin[64:].astype(bf16), b_lin,
               W1[:64].astype(bf16), W1[64:].astype(bf16), b1,
               W2, b2, W3, b3, W4, b4)

    outs = []
    for b in range(B):
        table = _build_table(pc0_map, pc1_map, flow_map, b, C, H * W)
        g = _sc_gather(table, idx[b], N_pad)
        outs.append(_mlp(g, N, weights))
    return jnp.stack(outs, axis=0)


# speedup vs baseline: 1.2760x; 1.0432x over previous
import functools

import jax
import jax.numpy as jnp
from jax import lax
from jax.experimental import pallas as pl
from jax.experimental.pallas import tpu as pltpu
from jax.experimental.pallas import tpu_sc as plsc

NC, NS = 2, 16
NW = NC * NS
GCH = 256
TD = 256
TDP = 128


def _sc_gather(table, idx, bn_pad):
    b_per_w = bn_pad // NW
    mesh = plsc.VectorSubcoreMesh(core_axis_name="c", subcore_axis_name="s")

    @functools.partial(
        pl.kernel,
        out_type=jax.ShapeDtypeStruct((bn_pad, TDP), jnp.int32),
        mesh=mesh,
        scratch_types=[
            pltpu.VMEM((b_per_w,), jnp.int32),
            pltpu.VMEM((GCH, TDP), jnp.int32),
            pltpu.VMEM((GCH, TDP), jnp.int32),
            pltpu.SemaphoreType.DMA,
            pltpu.SemaphoreType.DMA,
            pltpu.SemaphoreType.DMA,
            pltpu.SemaphoreType.DMA,
        ],
    )
    def gather_kernel(t_hbm, idx_hbm, g_hbm, idx_v, r0, r1, sg0, sg1,
                      sw0, sw1):
        wid = lax.axis_index("s") * NC + lax.axis_index("c")
        base = wid * b_per_w
        pltpu.sync_copy(idx_hbm.at[pl.ds(base, b_per_w)], idx_v)
        npairs = b_per_w // (2 * GCH)

        def gcp(buf, sem, chunk):
            off = chunk * GCH
            return pltpu.make_async_copy(
                t_hbm.at[idx_v.at[pl.ds(off, GCH)]], buf, sem)

        def wcp(buf, sem, chunk):
            off = chunk * GCH
            return pltpu.make_async_copy(
                buf, g_hbm.at[pl.ds(base + off, GCH)], sem)

        gcp(r0, sg0, 0).start()
        gcp(r1, sg1, 1).start()

        @pl.loop(0, npairs - 1)
        def _(i):
            c0 = 2 * i
            gcp(r0, sg0, c0).wait()
            wcp(r0, sw0, c0).start()
            gcp(r1, sg1, c0 + 1).wait()
            wcp(r1, sw1, c0 + 1).start()
            wcp(r0, sw0, c0).wait()
            gcp(r0, sg0, c0 + 2).start()
            wcp(r1, sw1, c0 + 1).wait()
            gcp(r1, sg1, c0 + 3).start()

        last = 2 * (npairs - 1)
        gcp(r0, sg0, last).wait()
        wcp(r0, sw0, last).start()
        gcp(r1, sg1, last + 1).wait()
        wcp(r1, sw1, last + 1).start()
        wcp(r0, sw0, last).wait()
        wcp(r1, sw1, last + 1).wait()

    return gather_kernel(table, idx)


def _table_body(p0, p1, fl, o_ref):
    bf = jnp.bfloat16
    xs = [m[0].astype(bf) for m in (p0, p1, fl)]
    X = jnp.concatenate(xs, axis=0)
    X = jnp.reshape(X, (192, X.shape[1] * X.shape[2]))
    ci = lax.broadcasted_iota(jnp.int32, (192, 128), 0)
    lj = lax.broadcasted_iota(jnp.int32, (192, 128), 1)
    p_ = ci // 64
    ch = ci % 64
    e_lo = ((ch < 32) & (lj == p_ * 32 + ch)).astype(bf)
    e_hi = ((ch >= 32) & (lj == p_ * 32 + ch - 32)).astype(bf)
    dn = (((0,), (0,)), ((), ()))
    lo = lax.dot_general(X, e_lo, dn, preferred_element_type=jnp.float32)
    hi = lax.dot_general(X, e_hi, dn, preferred_element_type=jnp.float32)
    lo_b = lax.bitcast_convert_type(lo, jnp.int32)
    hi_b = lax.bitcast_convert_type(hi, jnp.int32)
    o_ref[...] = (lax.shift_right_logical(lo_b, 16)
                  | (hi_b & jnp.int32(-65536)))


def _build_table(pc0, pc1, fl, b, C, HW):
    HB = 16
    T = HB * 512
    map_spec = pl.BlockSpec((1, C, HB, 512), lambda j: (b, 0, j, 0))
    return pl.pallas_call(
        _table_body,
        out_shape=jax.ShapeDtypeStruct((HW, TDP), jnp.int32),
        grid=(HW // T,),
        in_specs=[map_spec, map_spec, map_spec],
        out_specs=pl.BlockSpec((T, TDP), lambda j: (j, 0)),
        compiler_params=pltpu.CompilerParams(
            dimension_semantics=("parallel",)),
    )(pc0, pc1, fl)


def _gelu_exact(x):
    return 0.5 * x * (1.0 + lax.erf(x * 0.7071067811865476))


def _mlp_body(g, wl0, wl1, bl, w1a, w1b, b1r, w2r, b2r, w3r, b3r,
              w4r, b4r, o_ref):
    f32 = jnp.float32
    gb = g[...]

    def unpack(s):
        lo = lax.bitcast_convert_type(jnp.left_shift(s, 16), f32)
        hi = lax.bitcast_convert_type(s & jnp.int32(-65536), f32)
        return jnp.concatenate([lo, hi], axis=1).astype(jnp.bfloat16)

    g0 = unpack(gb[:, 0:32])
    g1 = unpack(gb[:, 32:64])
    g2 = unpack(gb[:, 64:96])
    x = (jnp.dot(g0, wl0[...], preferred_element_type=f32)
         + jnp.dot(g1, wl1[...], preferred_element_type=f32) + bl[...])
    h = (jnp.dot(x.astype(jnp.bfloat16), w1a[...],
                 preferred_element_type=f32)
         + jnp.dot(g2, w1b[...], preferred_element_type=f32) + b1r[...])
    h = _gelu_exact(h)
    h = _gelu_exact(jnp.dot(h, w2r[...], preferred_element_type=f32)
                    + b2r[...])
    h = _gelu_exact(jnp.dot(h, w3r[...], preferred_element_type=f32)
                    + b3r[...])
    s = jnp.sum(h * w4r[...][:, 0], axis=1) + b4r[0]
    o_ref[...] = jax.nn.sigmoid(s)


def _mlp(g, N, weights):
    BLK = 512
    full = lambda shape: pl.BlockSpec(shape, lambda i: tuple(0 for _ in shape))
    return pl.pallas_call(
        _mlp_body,
        out_shape=jax.ShapeDtypeStruct((N,), jnp.float32),
        grid=(N // BLK,),
        in_specs=[pl.BlockSpec((BLK, TDP), lambda i: (i, 0)),
                  full((64, 64)), full((64, 64)), full((64,)),
                  full((64, 64)), full((64, 64)), full((64,)),
                  full((64, 32)), full((32,)),
                  full((32, 16)), full((16,)),
                  full((16, 1)), full((1,))],
        out_specs=pl.BlockSpec((BLK,), lambda i: (i,)),
        compiler_params=pltpu.CompilerParams(
            dimension_semantics=("parallel",)),
    )(g, *weights)


def kernel(pc0_map, pc1_map, flow_map, lidar_voxel_coords, radar_voxel_coords,
           W_lin, b_lin, W1, b1, W2, b2, W3, b3, W4, b4):
    B, C, H, W = pc0_map.shape
    NL = lidar_voxel_coords.shape[1]
    NR = radar_voxel_coords.shape[1]
    N = NL + NR
    N_pad = -(-N // (NW * 2 * GCH)) * (NW * 2 * GCH)

    coords = jnp.concatenate([lidar_voxel_coords, radar_voxel_coords], axis=1)
    idx = (coords[..., 1] * W + coords[..., 2]).astype(jnp.int32)
    idx = jnp.pad(idx, ((0, 0), (0, N_pad - N)))

    bf16 = jnp.bfloat16
    weights = (W_lin[:64].astype(bf16), W_lin[64:].astype(bf16), b_lin,
               W1[:64].astype(bf16), W1[64:].astype(bf16), b1,
               W2, b2, W3, b3, W4, b4)

    outs = []
    for b in range(B):
        table = _build_table(pc0_map, pc1_map, flow_map, b, C, H * W)
        g = _sc_gather(table, idx[b], N_pad)
        outs.append(_mlp(g, N, weights))
    return jnp.stack(outs, axis=0)


# speedup vs baseline: 1.3026x; 1.0209x over previous
import functools

import jax
import jax.numpy as jnp
from jax import lax
from jax.experimental import pallas as pl
from jax.experimental.pallas import tpu as pltpu
from jax.experimental.pallas import tpu_sc as plsc

NC, NS = 2, 16
NW = NC * NS
GCH = 256
TD = 256
TDP = 128


def _sc_gather(table, idx, bn_pad):
    b_per_w = bn_pad // NW
    mesh = plsc.VectorSubcoreMesh(core_axis_name="c", subcore_axis_name="s")

    @functools.partial(
        pl.kernel,
        out_type=jax.ShapeDtypeStruct((bn_pad, TDP), jnp.int32),
        mesh=mesh,
        scratch_types=[
            pltpu.VMEM((b_per_w,), jnp.int32),
            pltpu.VMEM((GCH, TDP), jnp.int32),
            pltpu.VMEM((GCH, TDP), jnp.int32),
            pltpu.SemaphoreType.DMA,
            pltpu.SemaphoreType.DMA,
            pltpu.SemaphoreType.DMA,
            pltpu.SemaphoreType.DMA,
        ],
    )
    def gather_kernel(t_hbm, idx_hbm, g_hbm, idx_v, r0, r1, sg0, sg1,
                      sw0, sw1):
        wid = lax.axis_index("s") * NC + lax.axis_index("c")
        base = wid * b_per_w
        pltpu.sync_copy(idx_hbm.at[pl.ds(base, b_per_w)], idx_v)
        npairs = b_per_w // (2 * GCH)

        def gcp(buf, sem, chunk):
            off = chunk * GCH
            return pltpu.make_async_copy(
                t_hbm.at[idx_v.at[pl.ds(off, GCH)]], buf, sem)

        def wcp(buf, sem, chunk):
            off = chunk * GCH
            return pltpu.make_async_copy(
                buf, g_hbm.at[pl.ds(base + off, GCH)], sem)

        gcp(r0, sg0, 0).start()
        gcp(r1, sg1, 1).start()

        @pl.loop(0, npairs - 1)
        def _(i):
            c0 = 2 * i
            gcp(r0, sg0, c0).wait()
            wcp(r0, sw0, c0).start()
            gcp(r1, sg1, c0 + 1).wait()
            wcp(r1, sw1, c0 + 1).start()
            wcp(r0, sw0, c0).wait()
            gcp(r0, sg0, c0 + 2).start()
            wcp(r1, sw1, c0 + 1).wait()
            gcp(r1, sg1, c0 + 3).start()

        last = 2 * (npairs - 1)
        gcp(r0, sg0, last).wait()
        wcp(r0, sw0, last).start()
        gcp(r1, sg1, last + 1).wait()
        wcp(r1, sw1, last + 1).start()
        wcp(r0, sw0, last).wait()
        wcp(r1, sw1, last + 1).wait()

    return gather_kernel(table, idx)


def _table_body(p0, p1, fl, o_ref):
    bf = jnp.bfloat16
    xs = [m[0].astype(bf) for m in (p0, p1, fl)]
    X = jnp.concatenate(xs, axis=0)
    X = jnp.reshape(X, (192, X.shape[1] * X.shape[2]))
    ci = lax.broadcasted_iota(jnp.int32, (192, 128), 0)
    lj = lax.broadcasted_iota(jnp.int32, (192, 128), 1)
    p_ = ci // 64
    ch = ci % 64
    e_lo = ((ch < 32) & (lj == p_ * 32 + ch)).astype(bf)
    e_hi = ((ch >= 32) & (lj == p_ * 32 + ch - 32)).astype(bf)
    dn = (((0,), (0,)), ((), ()))
    lo = lax.dot_general(X, e_lo, dn, preferred_element_type=jnp.float32)
    hi = lax.dot_general(X, e_hi, dn, preferred_element_type=jnp.float32)
    lo_b = lax.bitcast_convert_type(lo, jnp.int32)
    hi_b = lax.bitcast_convert_type(hi, jnp.int32)
    o_ref[...] = (lax.shift_right_logical(lo_b, 16)
                  | (hi_b & jnp.int32(-65536)))


def _build_table(pc0, pc1, fl, b, C, HW):
    HB = 32
    T = HB * 512
    map_spec = pl.BlockSpec((1, C, HB, 512), lambda j: (b, 0, j, 0))
    return pl.pallas_call(
        _table_body,
        out_shape=jax.ShapeDtypeStruct((HW, TDP), jnp.int32),
        grid=(HW // T,),
        in_specs=[map_spec, map_spec, map_spec],
        out_specs=pl.BlockSpec((T, TDP), lambda j: (j, 0)),
        compiler_params=pltpu.CompilerParams(
            dimension_semantics=("parallel",)),
    )(pc0, pc1, fl)


def _gelu_exact(x):
    return 0.5 * x * (1.0 + lax.erf(x * 0.7071067811865476))


def _mlp_body(g, wl0, wl1, bl, w1a, w1b, b1r, w2r, b2r, w3r, b3r,
              w4r, b4r, o_ref):
    f32 = jnp.float32
    gb = g[...]

    def unpack(s):
        lo = lax.bitcast_convert_type(jnp.left_shift(s, 16), f32)
        hi = lax.bitcast_convert_type(s & jnp.int32(-65536), f32)
        return jnp.concatenate([lo, hi], axis=1).astype(jnp.bfloat16)

    g0 = unpack(gb[:, 0:32])
    g1 = unpack(gb[:, 32:64])
    g2 = unpack(gb[:, 64:96])
    x = (jnp.dot(g0, wl0[...], preferred_element_type=f32)
         + jnp.dot(g1, wl1[...], preferred_element_type=f32) + bl[...])
    h = (jnp.dot(x.astype(jnp.bfloat16), w1a[...],
                 preferred_element_type=f32)
         + jnp.dot(g2, w1b[...], preferred_element_type=f32) + b1r[...])
    h = _gelu_exact(h)
    h = _gelu_exact(jnp.dot(h, w2r[...], preferred_element_type=f32)
                    + b2r[...])
    h = _gelu_exact(jnp.dot(h, w3r[...], preferred_element_type=f32)
                    + b3r[...])
    s = jnp.sum(h * w4r[...][:, 0], axis=1) + b4r[0]
    o_ref[...] = jax.nn.sigmoid(s)


def _mlp(g, N, weights):
    BLK = 512
    full = lambda shape: pl.BlockSpec(shape, lambda i: tuple(0 for _ in shape))
    return pl.pallas_call(
        _mlp_body,
        out_shape=jax.ShapeDtypeStruct((N,), jnp.float32),
        grid=(N // BLK,),
        in_specs=[pl.BlockSpec((BLK, TDP), lambda i: (i, 0)),
                  full((64, 64)), full((64, 64)), full((64,)),
                  full((64, 64)), full((64, 64)), full((64,)),
                  full((64, 32)), full((32,)),
                  full((32, 16)), full((16,)),
                  full((16, 1)), full((1,))],
        out_specs=pl.BlockSpec((BLK,), lambda i: (i,)),
        compiler_params=pltpu.CompilerParams(
            dimension_semantics=("parallel",)),
    )(g, *weights)


def kernel(pc0_map, pc1_map, flow_map, lidar_voxel_coords, radar_voxel_coords,
           W_lin, b_lin, W1, b1, W2, b2, W3, b3, W4, b4):
    B, C, H, W = pc0_map.shape
    NL = lidar_voxel_coords.shape[1]
    NR = radar_voxel_coords.shape[1]
    N = NL + NR
    N_pad = -(-N // (NW * 2 * GCH)) * (NW * 2 * GCH)

    coords = jnp.concatenate([lidar_voxel_coords, radar_voxel_coords], axis=1)
    idx = (coords[..., 1] * W + coords[..., 2]).astype(jnp.int32)
    idx = jnp.pad(idx, ((0, 0), (0, N_pad - N)))

    bf16 = jnp.bfloat16
    weights = (W_lin[:64].astype(bf16), W_lin[64:].astype(bf16), b_lin,
               W1[:64].astype(bf16), W1[64:].astype(bf16), b1,
               W2, b2, W3, b3, W4, b4)

    outs = []
    for b in range(B):
        table = _build_table(pc0_map, pc1_map, flow_map, b, C, H * W)
        g = _sc_gather(table, idx[b], N_pad)
        outs.append(_mlp(g, N, weights))
    return jnp.stack(outs, axis=0)
